# baseline (speedup 1.0000x reference)
import jax
import jax.numpy as jnp
from jax.experimental import pallas as pl
from jax.experimental.pallas import tpu as pltpu

_V, _D, _M, _K = 50000, 512, 4, 512
_CHUNK = _D // _M
_TV = 1000
_NBUF = 5
_NOBUF = 3


def _dpq_tile_kernel(logits_hbm, cb_ref, out_hbm, buf, obuf, sem, osem):
    i = pl.program_id(0)
    n = pl.num_programs(0)

    def in_dma(chunk, slot, m):
        return pltpu.make_async_copy(
            logits_hbm.at[pl.ds(chunk * _TV, _TV), m],
            buf.at[slot, m],
            sem.at[slot],
        )

    def out_dma(chunk, slot):
        return pltpu.make_async_copy(
            obuf.at[slot],
            out_hbm.at[pl.ds(chunk * _TV, _TV)],
            osem.at[slot],
        )

    @pl.when(i == 0)
    def _():
        for s in range(_NBUF):
            for m in range(_M):
                in_dma(s, s, m).start()

    slot = jax.lax.rem(i, _NBUF)
    for m in range(_M):
        in_dma(i, slot, m).wait()

    oslot = jax.lax.rem(i, _NOBUF)

    @pl.when(i >= _NOBUF)
    def _():
        out_dma(i - _NOBUF, oslot).wait()

    for m in range(_M):
        x = buf[slot, m]
        x = x - jnp.max(x, axis=-1, keepdims=True)
        e = jnp.exp(x)
        attn = e / jnp.sum(e, axis=-1, keepdims=True)
        obuf[oslot, :, m * _CHUNK:(m + 1) * _CHUNK] = jnp.dot(
            attn, cb_ref[m], preferred_element_type=jnp.float32
        )

    out_dma(i, oslot).start()

    @pl.when(i + _NBUF < n)
    def _():
        for m in range(_M):
            in_dma(i + _NBUF, slot, m).start()

    @pl.when(i == n - 1)
    def _():
        for s in range(_NOBUF):
            out_dma(0, s).wait()


def kernel(assign_logits, codebooks):
    return pl.pallas_call(
        _dpq_tile_kernel,
        grid=(_V // _TV,),
        in_specs=[
            pl.BlockSpec(memory_space=pl.ANY),
            pl.BlockSpec((_M, _K, _CHUNK), lambda i: (0, 0, 0)),
        ],
        out_specs=pl.BlockSpec(memory_space=pl.ANY),
        out_shape=jax.ShapeDtypeStruct((_V, _D), jnp.float32),
        scratch_shapes=[
            pltpu.VMEM((_NBUF, _M, _TV, _K), jnp.float32),
            pltpu.VMEM((_NOBUF, _TV, _D), jnp.float32),
            pltpu.SemaphoreType.DMA((_NBUF,)),
            pltpu.SemaphoreType.DMA((_NOBUF,)),
        ],
    )(assign_logits, codebooks)


# speedup vs baseline: 1.0064x; 1.0064x over previous
import jax
import jax.numpy as jnp
from jax.experimental import pallas as pl
from jax.experimental.pallas import tpu as pltpu

_V, _D, _M, _K = 50000, 512, 4, 512
_CHUNK = _D // _M
_TV = 800
_NBUF = 6
_NOBUF = 3


def _dpq_tile_kernel(logits_hbm, cb_ref, out_hbm, buf, obuf, sem, osem):
    i = pl.program_id(0)
    n = pl.num_programs(0)

    def in_dma(chunk, slot, m):
        return pltpu.make_async_copy(
            logits_hbm.at[pl.ds(chunk * _TV, _TV), m],
            buf.at[slot, m],
            sem.at[slot],
        )

    def out_dma(chunk, slot):
        return pltpu.make_async_copy(
            obuf.at[slot],
            out_hbm.at[pl.ds(chunk * _TV, _TV)],
            osem.at[slot],
        )

    @pl.when(i == 0)
    def _():
        for s in range(_NBUF):
            for m in range(_M):
                in_dma(s, s, m).start()

    slot = jax.lax.rem(i, _NBUF)
    for m in range(_M):
        in_dma(i, slot, m).wait()

    oslot = jax.lax.rem(i, _NOBUF)

    @pl.when(i >= _NOBUF)
    def _():
        out_dma(i - _NOBUF, oslot).wait()

    for m in range(_M):
        x = buf[slot, m]
        x = x - jnp.max(x, axis=-1, keepdims=True)
        e = jnp.exp(x)
        attn = e / jnp.sum(e, axis=-1, keepdims=True)
        obuf[oslot, :, m * _CHUNK:(m + 1) * _CHUNK] = jnp.dot(
            attn, cb_ref[m], preferred_element_type=jnp.float32
        )

    out_dma(i, oslot).start()

    @pl.when(i + _NBUF < n)
    def _():
        for m in range(_M):
            in_dma(i + _NBUF, slot, m).start()

    @pl.when(i == n - 1)
    def _():
        for s in range(_NOBUF):
            out_dma(0, s).wait()


def kernel(assign_logits, codebooks):
    return pl.pallas_call(
        _dpq_tile_kernel,
        grid=(_V // _TV,),
        in_specs=[
            pl.BlockSpec(memory_space=pl.ANY),
            pl.BlockSpec((_M, _K, _CHUNK), lambda i: (0, 0, 0)),
        ],
        out_specs=pl.BlockSpec(memory_space=pl.ANY),
        out_shape=jax.ShapeDtypeStruct((_V, _D), jnp.float32),
        scratch_shapes=[
            pltpu.VMEM((_NBUF, _M, _TV, _K), jnp.float32),
            pltpu.VMEM((_NOBUF, _TV, _D), jnp.float32),
            pltpu.SemaphoreType.DMA((_NBUF,)),
            pltpu.SemaphoreType.DMA((_NOBUF,)),
        ],
    )(assign_logits, codebooks)


# speedup vs baseline: 1.0081x; 1.0017x over previous
import jax
import jax.numpy as jnp
from jax.experimental import pallas as pl
from jax.experimental.pallas import tpu as pltpu

_V, _D, _M, _K = 50000, 512, 4, 512
_CHUNK = _D // _M
_TV = 400
_NBUF = 8
_NOBUF = 4

assert _V % _TV == 0 and _TV % 8 == 0


def _dpq_tile_kernel(logits_hbm, cb_ref, out_hbm, buf, obuf, sem, osem):
    i = pl.program_id(0)
    n = pl.num_programs(0)

    def in_dma(chunk, slot, m):
        return pltpu.make_async_copy(
            logits_hbm.at[pl.ds(chunk * _TV, _TV), m],
            buf.at[slot, m],
            sem.at[slot],
        )

    def out_dma(chunk, slot):
        return pltpu.make_async_copy(
            obuf.at[slot],
            out_hbm.at[pl.ds(chunk * _TV, _TV)],
            osem.at[slot],
        )

    @pl.when(i == 0)
    def _():
        for s in range(_NBUF):
            for m in range(_M):
                in_dma(s, s, m).start()

    slot = jax.lax.rem(i, _NBUF)
    for m in range(_M):
        in_dma(i, slot, m).wait()

    oslot = jax.lax.rem(i, _NOBUF)

    @pl.when(i >= _NOBUF)
    def _():
        out_dma(i - _NOBUF, oslot).wait()

    for m in range(_M):
        x = buf[slot, m]
        x = x - jnp.max(x, axis=-1, keepdims=True)
        e = jnp.exp(x)
        attn = e / jnp.sum(e, axis=-1, keepdims=True)
        obuf[oslot, :, m * _CHUNK:(m + 1) * _CHUNK] = jnp.dot(
            attn, cb_ref[m], preferred_element_type=jnp.float32
        )

    out_dma(i, oslot).start()

    @pl.when(i + _NBUF < n)
    def _():
        for m in range(_M):
            in_dma(i + _NBUF, slot, m).start()

    @pl.when(i == n - 1)
    def _():
        for s in range(_NOBUF):
            out_dma(0, s).wait()


def kernel(assign_logits, codebooks):
    return pl.pallas_call(
        _dpq_tile_kernel,
        grid=(_V // _TV,),
        in_specs=[
            pl.BlockSpec(memory_space=pl.ANY),
            pl.BlockSpec((_M, _K, _CHUNK), lambda i: (0, 0, 0)),
        ],
        out_specs=pl.BlockSpec(memory_space=pl.ANY),
        out_shape=jax.ShapeDtypeStruct((_V, _D), jnp.float32),
        scratch_shapes=[
            pltpu.VMEM((_NBUF, _M, _TV, _K), jnp.float32),
            pltpu.VMEM((_NOBUF, _TV, _D), jnp.float32),
            pltpu.SemaphoreType.DMA((_NBUF,)),
            pltpu.SemaphoreType.DMA((_NOBUF,)),
        ],
    )(assign_logits, codebooks)
